# baseline (speedup 1.0000x reference)
import jax
import jax.numpy as jnp
from jax import lax
from jax.experimental import pallas as pl
from jax.experimental.pallas import tpu as pltpu
from jax.experimental.pallas import tpu_sc as plsc

NC = 2
NS = 16
LANES = 16

CHUNK = 128
NBUF = 2


def _build_step(npad, dh, epad):
    edges_per_tile = epad // NS
    chunks_per_tile = edges_per_tile // CHUNK
    rpt = npad // NS
    f32 = jnp.float32

    def body(x_ref, e_ref, v_ref, zero_ref, y_ref,
             acc_ref, ebuf, vbuf, r0, r1, gsem, ssem):
        rowbufs = (r0, r1)
        c = lax.axis_index("c")
        s = lax.axis_index("s")
        gbase = s * chunks_per_tile

        pltpu.sync_copy(zero_ref.at[pl.ds(s * rpt, rpt)],
                        acc_ref.at[pl.ds(s * rpt, rpt)])
        plsc.subcore_barrier()

        ngroups = chunks_per_tile // NBUF

        @pl.loop(0, ngroups)
        def _edges(t):
            g0 = t * NBUF
            pltpu.sync_copy(e_ref.at[c, pl.ds(gbase + g0, NBUF)], ebuf)
            pltpu.sync_copy(v_ref.at[s * ngroups + t], vbuf)
            gh = [pltpu.async_copy(x_ref.at[ebuf.at[b, 0]], rowbufs[b],
                                   gsem)
                  for b in range(NBUF)]
            sh = []
            for b in range(NBUF):
                gh[b].wait()
                rows = rowbufs[b]

                @pl.loop(0, CHUNK // LANES)
                def _scale(eg, b=b, rows=rows):
                    wv = vbuf[b, pl.ds(eg * LANES, LANES)]
                    for j in range(LANES):
                        w = wv[j]
                        e = eg * LANES + j
                        for v in range(dh // LANES):
                            sl = pl.ds(v * LANES, LANES)
                            rows[e, sl] = rows[e, sl] * w

                sh.append(pltpu.async_copy(rows, acc_ref.at[ebuf.at[b, 1]],
                                           ssem, add=True))
            for h in sh:
                h.wait()

        plsc.subcore_barrier()

        pltpu.sync_copy(acc_ref.at[pl.ds(s * rpt, rpt)],
                        y_ref.at[pl.ds(c * npad + s * rpt, rpt)])

    return pl.kernel(
        body,
        out_type=jax.ShapeDtypeStruct((NC * npad, dh), f32),
        mesh=plsc.VectorSubcoreMesh(core_axis_name="c", subcore_axis_name="s"),
        scratch_types=[
            pltpu.VMEM_SHARED((npad, dh), f32),
            pltpu.VMEM((NBUF, 2, CHUNK), jnp.int32),
            pltpu.VMEM((NBUF, CHUNK), f32),
            pltpu.VMEM((CHUNK, dh), f32),
            pltpu.VMEM((CHUNK, dh), f32),
            pltpu.SemaphoreType.DMA,
            pltpu.SemaphoreType.DMA,
        ],
    )


def _pick_block(n):
    for blk in range(2048, 0, -1):
        if n % blk == 0:
            return blk
    return n


def _tc_add(a, b):
    n, d = a.shape
    blk = _pick_block(n)

    def body(a_ref, b_ref, o_ref):
        o_ref[...] = a_ref[...] + b_ref[...]

    return pl.pallas_call(
        body,
        out_shape=jax.ShapeDtypeStruct((n, d), a.dtype),
        grid=(n // blk,),
        in_specs=[pl.BlockSpec((blk, d), lambda i: (i, 0))] * 2,
        out_specs=pl.BlockSpec((blk, d), lambda i: (i, 0)),
    )(a, b)


def _tc_scale(a, scale):
    n, d = a.shape
    blk = _pick_block(n)

    def body(s_ref, a_ref, o_ref):
        o_ref[...] = a_ref[...] * s_ref[0]

    return pl.pallas_call(
        body,
        out_shape=jax.ShapeDtypeStruct((n, d), a.dtype),
        grid=(n // blk,),
        in_specs=[
            pl.BlockSpec(memory_space=pltpu.SMEM),
            pl.BlockSpec((blk, d), lambda i: (i, 0)),
        ],
        out_specs=pl.BlockSpec((blk, d), lambda i: (i, 0)),
    )(jnp.reshape(scale, (1,)).astype(a.dtype), a)


def kernel(E_user, E_item, adj_indices, adj_values, K):
    n_users, d = E_user.shape
    n_items = E_item.shape[0]
    n = n_users + n_items
    dh = d // NC
    nnz = adj_values.shape[0]

    npad = -(-n // (NS * 8)) * (NS * 8)
    ealign = NS * CHUNK * 8
    epad = -(-nnz // ealign) * ealign

    x0 = jnp.concatenate([E_user, E_item], axis=0)
    xs = jnp.concatenate(
        [jnp.pad(x0[:, c * dh:(c + 1) * dh], ((0, npad - n), (0, 0)))
         for c in range(NC)], axis=0)

    nchunks = epad // CHUNK
    dst = jnp.pad(adj_indices[0], (0, epad - nnz)).reshape(nchunks, CHUNK)
    src = jnp.pad(adj_indices[1], (0, epad - nnz))
    val = jnp.pad(adj_values, (0, epad - nnz)).reshape(nchunks // NBUF,
                                                       NBUF, CHUNK)
    packed = jnp.stack(
        [jnp.stack([(src + c * npad).reshape(nchunks, CHUNK), dst], axis=1)
         for c in range(NC)], axis=0)

    zeros = jnp.zeros((npad, dh), jnp.float32)
    step = _build_step(npad, dh, epad)

    def body(_, carry):
        x, acc = carry
        y = step(x, packed, val, zeros)
        return (y, _tc_add(acc, y))

    _, acc = lax.fori_loop(0, K, body, (xs, xs))
    acc = _tc_scale(acc, 1.0 / (K + 1.0))
    accf = jnp.concatenate([acc[:n], acc[npad:npad + n]], axis=1)
    return accf[:n_users], accf[n_users:]
